# baseline (speedup 1.0000x reference)
import functools

import numpy as np

import jax
import jax.numpy as jnp
from jax import lax
from jax.experimental import pallas as pl
from jax.experimental.pallas import tpu as pltpu
from jax.experimental.pallas import tpu_sc as plsc

N_ENT = 10000
N_REL = 10000
IN_DIM = 128
OUT_DIM = 256
MARGIN = 6.0
EPSILON = 2.0
NEG_RATE = 10
N_TRIPLES = 110000

NPAD = 110592
CHUNK = NPAD // 32
TRASH = 10000
TBL_ROWS = 10240
HALF = 5000
ACC_ROWS = 2560
QUARTER = 2500
ACC_W = 272

_mesh = plsc.VectorSubcoreMesh(core_axis_name="c", subcore_axis_name="s")


_GDN = lax.GatherDimensionNumbers(offset_dims=(), collapsed_slice_dims=(0,),
                                 start_index_map=(0,))


def _allsum(v):
    lanes = lax.iota(jnp.int32, 16)
    for k in (1, 2, 4, 8):
        p = (lanes ^ k).reshape(16, 1)
        v = v + lax.gather(v, p, _GDN, (1,),
                           mode=lax.GatherScatterMode.PROMISE_IN_BOUNDS)
    return v


def _f32(shape):
    return jax.ShapeDtypeStruct(shape, jnp.float32)


@functools.partial(
    pl.kernel,
    mesh=_mesh,
    out_type=[_f32((2, TBL_ROWS, 128)), _f32((2, TBL_ROWS, 128))],
    scratch_types=[
        pltpu.VMEM((128, 128), jnp.float32),
        pltpu.VMEM((128, 128), jnp.float32),
        pltpu.VMEM((128,), jnp.int32),
        pltpu.VMEM_SHARED((TBL_ROWS, 128), jnp.float32),
    ],
)
def _k1_hist(seg_hbm, rel_hbm, ones_hbm, zeros_hbm, out_e, out_r,
             stage_v, gbuf, idx_v, hist):
    cid = lax.axis_index("c")
    sid = lax.axis_index("s")
    wid = cid * 16 + sid
    lanes = lax.iota(jnp.int32, 16)
    pltpu.sync_copy(ones_hbm, stage_v)
    pltpu.sync_copy(zeros_hbm, gbuf)

    def ident(b, _):
        def w(k, _):
            idx_v[pl.ds(k * 16, 16)] = lanes + (sid * 640 + b * 128 + k * 16)
            return 0
        lax.fori_loop(0, 8, w, 0)
        return 0

    def run(src_hbm, base, nb, out):
        def zinit(b, _):
            ident(b, 0)
            pltpu.sync_copy(gbuf, hist.at[idx_v])
            return 0
        lax.fori_loop(0, 5, zinit, 0)
        plsc.subcore_barrier()

        def body(b, _):
            pltpu.sync_copy(src_hbm.at[pl.ds(base + b * 128, 128)], idx_v)
            pltpu.sync_copy(stage_v, hist.at[idx_v], add=True)
            return 0
        lax.fori_loop(0, nb, body, 0)
        plsc.subcore_barrier()

        def dump(b, _):
            ident(b, 0)
            pltpu.sync_copy(hist.at[idx_v], gbuf)
            pltpu.sync_copy(gbuf, out.at[cid, pl.ds(sid * 640 + b * 128, 128)])
            return 0
        lax.fori_loop(0, 5, dump, 0)
        plsc.subcore_barrier()
        pltpu.sync_copy(zeros_hbm, gbuf)

    run(seg_hbm, wid * (2 * CHUNK), (2 * CHUNK) // 128, out_e)
    run(rel_hbm, wid * CHUNK, CHUNK // 128, out_r)


def _k2a_body(he_ref, hr_ref, e_ref, r_ref, aw_ref, ab_ref, g0_ref, b0_ref,
              wp_ref, biasp_ref, mean1_ref, cntr_ref):
    he = he_ref[...]
    hr = hr_ref[...]
    cnt_e = he[0, :N_ENT, 0] + he[1, :N_ENT, 0]
    cnt_r_full = hr[0, :, 0] + hr[1, :, 0]
    cnt_r = cnt_r_full[:N_REL]
    E = e_ref[...]
    R = r_ref[...]
    dot = functools.partial(lax.dot_general,
                            preferred_element_type=jnp.float32,
                            precision=lax.Precision.HIGHEST)
    ce = cnt_e.reshape(1, N_ENT)
    cr = cnt_r.reshape(1, N_REL)
    sE = dot(ce, E, (((1,), (0,)), ((), ())))[0]
    qE = dot(ce, E * E, (((1,), (0,)), ((), ())))[0]
    qR = dot(cr, R * R, (((1,), (0,)), ((), ())))[0]
    n2 = jnp.float32(2 * N_TRIPLES)
    mE = sE / n2
    vE = qE / n2 - mE * mE
    vR = qR / jnp.float32(N_TRIPLES)
    m = jnp.concatenate([mE, mE, jnp.zeros_like(mE)])
    v = jnp.concatenate([vE, vE, vR])
    g0 = g0_ref[...]
    b0 = b0_ref[...]
    scale0 = g0 * lax.rsqrt(v + 1e-5)
    shift0 = b0 - m * scale0
    aw = aw_ref[...]
    wp_ref[...] = aw * scale0[None, :]
    biasp_ref[...] = dot(aw, shift0.reshape(3 * IN_DIM, 1),
                         (((1,), (0,)), ((), ())))[:, 0] + ab_ref[...]
    mean1_ref[...] = dot(aw, b0.reshape(3 * IN_DIM, 1),
                         (((1,), (0,)), ((), ())))[:, 0] + ab_ref[...]
    cntr_ref[...] = cnt_r_full


_k2a = pl.pallas_call(
    _k2a_body,
    out_shape=[_f32((OUT_DIM, 3 * IN_DIM)), _f32((OUT_DIM,)),
               _f32((OUT_DIM,)), _f32((TBL_ROWS,))],
)


def _k2b_body(e_ref, r_ref, wp_ref, pent_ref, p3_ref):
    wp = wp_ref[...]
    w1 = wp[:, :IN_DIM]
    w2 = wp[:, IN_DIM:2 * IN_DIM]
    w3 = wp[:, 2 * IN_DIM:]
    dot = functools.partial(lax.dot_general,
                            preferred_element_type=jnp.float32,
                            precision=lax.Precision.HIGHEST)
    E = e_ref[...]
    p1 = dot(E, w1, (((1,), (1,)), ((), ())))
    p2 = dot(E, w2, (((1,), (1,)), ((), ())))
    pent_ref[...] = jnp.concatenate([p1, p2], axis=1)
    p3_ref[...] = dot(r_ref[...], w3, (((1,), (1,)), ((), ())))


_k2b = pl.pallas_call(
    _k2b_body,
    grid=(10,),
    in_specs=[
        pl.BlockSpec((1024, IN_DIM), lambda i: (i, 0)),
        pl.BlockSpec((1024, IN_DIM), lambda i: (i, 0)),
        pl.BlockSpec((OUT_DIM, 3 * IN_DIM), lambda i: (0, 0)),
    ],
    out_specs=[
        pl.BlockSpec((1024, 2 * OUT_DIM), lambda i: (i, 0)),
        pl.BlockSpec((1024, OUT_DIM), lambda i: (i, 0)),
    ],
    out_shape=[_f32((TBL_ROWS, 2 * OUT_DIM)), _f32((TBL_ROWS, OUT_DIM))],
)


_GB = 16


@functools.partial(
    pl.kernel,
    mesh=_mesh,
    out_type=[_f32((2 * NPAD * OUT_DIM,)), _f32((32, OUT_DIM))],
    scratch_types=[
        pltpu.VMEM((CHUNK,), jnp.int32),
        pltpu.VMEM((CHUNK,), jnp.int32),
        pltpu.VMEM((CHUNK,), jnp.int32),
        pltpu.VMEM((_GB, 2 * OUT_DIM), jnp.float32),
        pltpu.VMEM((_GB, 2 * OUT_DIM), jnp.float32),
        pltpu.VMEM((_GB, OUT_DIM), jnp.float32),
        pltpu.VMEM((_GB * OUT_DIM,), jnp.float32),
        pltpu.VMEM((_GB * OUT_DIM,), jnp.float32),
        pltpu.VMEM((OUT_DIM,), jnp.float32),
        pltpu.VMEM((OUT_DIM,), jnp.float32),
        pltpu.SemaphoreType.DMA,
    ],
)
def _k3_edges(s_hbm, d_hbm, r_hbm, pent_hbm, p3_hbm, biasp_hbm,
              c_out, ssq_out, s_v, d_v, r_v, srows, drows, rrows,
              c1b, c2b, bias_v, ssq_v, sem):
    cid = lax.axis_index("c")
    sid = lax.axis_index("s")
    wid = cid * 16 + sid
    base = wid * CHUNK
    pltpu.sync_copy(s_hbm.at[pl.ds(base, CHUNK)], s_v)
    pltpu.sync_copy(d_hbm.at[pl.ds(base, CHUNK)], d_v)
    pltpu.sync_copy(r_hbm.at[pl.ds(base, CHUNK)], r_v)
    pltpu.sync_copy(biasp_hbm, bias_v)
    biasvs = [bias_v[pl.ds(j * 16, 16)] for j in range(16)]
    nb = CHUNK // _GB

    def batch(b, acc):
        o = b * _GB
        h1 = pltpu.async_copy(pent_hbm.at[s_v.at[pl.ds(o, _GB)]], srows, sem)
        h2 = pltpu.async_copy(pent_hbm.at[d_v.at[pl.ds(o, _GB)]], drows, sem)
        h3 = pltpu.async_copy(p3_hbm.at[r_v.at[pl.ds(o, _GB)]], rrows, sem)
        h1.wait()
        h2.wait()
        h3.wait()

        def row(i, acc):
            acc = list(acc)
            for j in range(16):
                j16 = j * 16
                s_lo = srows[i, pl.ds(j16, 16)]
                s_hi = srows[i, pl.ds(OUT_DIM + j16, 16)]
                d_lo = drows[i, pl.ds(j16, 16)]
                d_hi = drows[i, pl.ds(OUT_DIM + j16, 16)]
                rr = rrows[i, pl.ds(j16, 16)]
                c1 = s_lo + d_hi + rr + biasvs[j]
                c2 = d_lo + s_hi - rr + biasvs[j]
                c1b[pl.ds(i * OUT_DIM + j16, 16)] = c1
                c2b[pl.ds(i * OUT_DIM + j16, 16)] = c2
                acc[j] = acc[j] + c1 * c1 + c2 * c2
            return tuple(acc)

        acc = lax.fori_loop(0, _GB, row, acc)
        off = (base + o) * OUT_DIM
        pltpu.sync_copy(c1b, c_out.at[pl.ds(off, _GB * OUT_DIM)])
        pltpu.sync_copy(c2b, c_out.at[pl.ds(NPAD * OUT_DIM + off,
                                            _GB * OUT_DIM)])
        return acc

    zero = jnp.zeros((16,), jnp.float32)
    acc = lax.fori_loop(0, nb, batch, (zero,) * 16)
    for j in range(16):
        ssq_v[pl.ds(j * 16, 16)] = acc[j]
    pltpu.sync_copy(ssq_v, ssq_out.at[wid])


_SB = 32


def _make_k5(total_rows, qidx):
    nb = total_rows // 16 // _SB
    hw = OUT_DIM // 2
    dump = ACC_ROWS // 16

    @functools.partial(
        pl.kernel,
        mesh=_mesh,
        out_type=[_f32((2, ACC_ROWS, hw)), _f32((2, ACC_ROWS, hw)),
                  _f32((2, ACC_ROWS, hw))],
        scratch_types=[
            pltpu.VMEM((_SB * OUT_DIM,), jnp.float32),
            pltpu.VMEM((_SB, hw), jnp.float32),
            pltpu.VMEM((_SB, hw), jnp.float32),
            pltpu.VMEM((_SB, hw), jnp.float32),
            pltpu.VMEM((_SB, hw), jnp.float32),
            pltpu.VMEM((_SB,), jnp.int32),
            pltpu.VMEM((_SB,), jnp.int32),
            pltpu.VMEM((OUT_DIM,), jnp.float32),
            pltpu.VMEM((32,), jnp.float32),
            pltpu.VMEM_SHARED((ACC_ROWS, hw), jnp.float32),
            pltpu.VMEM_SHARED((ACC_ROWS, hw), jnp.float32),
            pltpu.VMEM_SHARED((ACC_ROWS, hw), jnp.float32),
        ],
    )
    def k5(c_hbm, seg_hbm, w2f_hbm, ucv_hbm, zeros_hbm, out_a, out_b, out_c,
           cbuf, stga, stgb, stgc, zbuf, segraw, idx_v, w_v, uc_v,
           acca, accb, accc):
        cid = lax.axis_index("c")
        sid = lax.axis_index("s")
        lanes = lax.iota(jnp.int32, 16)
        pltpu.sync_copy(w2f_hbm, w_v)
        pltpu.sync_copy(ucv_hbm, uc_v)
        pltpu.sync_copy(zeros_hbm, zbuf)
        pltpu.sync_copy(zeros_hbm, stgc)
        lane0 = uc_v[pl.ds(16, 16)]

        def ident(b, _):
            def w(k, _):
                idx_v[pl.ds(k * 16, 16)] = lanes + (sid * dump + b * _SB
                                                    + k * 16)
                return 0
            lax.fori_loop(0, _SB // 16, w, 0)
            return 0

        def zinit(b, _):
            ident(b, 0)
            pltpu.sync_copy(zbuf, acca.at[idx_v])
            pltpu.sync_copy(zbuf, accb.at[idx_v])
            pltpu.sync_copy(zbuf, accc.at[idx_v])
            return 0
        lax.fori_loop(0, dump // _SB, zinit, 0)
        plsc.subcore_barrier()
        ws = [w_v[pl.ds(j * 16, 16)] for j in range(16)]
        ucv = uc_v[pl.ds(0, 16)]
        lo = cid * HALF + qidx * QUARTER
        rows_per = total_rows // 16

        def batch(b, _):
            row0 = sid * rows_per + b * _SB
            pltpu.sync_copy(c_hbm.at[pl.ds(row0 * OUT_DIM,
                                           _SB * OUT_DIM)], cbuf)
            pltpu.sync_copy(seg_hbm.at[pl.ds(row0, _SB)], segraw)
            for k in range(_SB // 16):
                vseg = segraw[pl.ds(k * 16, 16)] - lo
                ok = (vseg >= 0) & (vseg < QUARTER)
                idx_v[pl.ds(k * 16, 16)] = jnp.where(ok, vseg, QUARTER)

            for i in range(_SB):
                cs = [cbuf[pl.ds(i * OUT_DIM + j * 16, 16)]
                      for j in range(16)]
                ua = cs[0] * ws[0]
                for j in range(1, 16):
                    ua = ua + cs[j] * ws[j]
                u = _allsum(ua) + ucv
                lk = jnp.maximum(u, 0.0) + 0.01 * jnp.minimum(u, 0.0)
                e = jnp.exp(-lk)
                for j in range(8):
                    stga[i, pl.ds(j * 16, 16)] = cs[j] * e
                for j in range(8):
                    stgb[i, pl.ds(j * 16, 16)] = cs[8 + j] * e
                stgc[i, pl.ds(0, 16)] = e * lane0

            pltpu.sync_copy(stga, acca.at[idx_v], add=True)
            pltpu.sync_copy(stgb, accb.at[idx_v], add=True)
            pltpu.sync_copy(stgc, accc.at[idx_v], add=True)
            return 0

        lax.fori_loop(0, nb, batch, 0)
        plsc.subcore_barrier()

        def dumploop(b, _):
            ident(b, 0)
            osl = pl.ds(sid * dump + b * _SB, _SB)
            pltpu.sync_copy(acca.at[idx_v], zbuf)
            pltpu.sync_copy(zbuf, out_a.at[cid, osl])
            pltpu.sync_copy(accb.at[idx_v], zbuf)
            pltpu.sync_copy(zbuf, out_b.at[cid, osl])
            pltpu.sync_copy(accc.at[idx_v], zbuf)
            pltpu.sync_copy(zbuf, out_c.at[cid, osl])
            return 0
        lax.fori_loop(0, dump // _SB, dumploop, 0)

    return k5


_k5_ent = [_make_k5(2 * NPAD, q) for q in (0, 1)]
_k5_rel = [_make_k5(NPAD, q) for q in (0, 1)]


def _k6_body(hs_ref, ebs_ref, rhs_ref, ebr_ref, cntr_ref, al_ref, de_ref,
             ew_ref, ebb_ref, rw_ref, rbb_ref, ente_ref, rtab_ref):
    alpha = al_ref[...]
    delta = de_ref[...]
    dot = functools.partial(lax.dot_general,
                            preferred_element_type=jnp.float32,
                            precision=lax.Precision.HIGHEST)
    ebs = ebs_ref[...]
    num = alpha[None, :] * hs_ref[...] + delta[None, :] * ebs[:, None]
    h_ent = num / jnp.where(ebs == 0.0, 1e-12, ebs)[:, None]
    ente_ref[...] = dot(h_ent, ew_ref[...],
                        (((1,), (1,)), ((), ()))) + ebb_ref[...][None, :]
    ebr = ebr_ref[...]
    rnum = alpha[None, :] * rhs_ref[...] + delta[None, :] * ebr[:, None]
    h_rel = rnum / jnp.maximum(cntr_ref[...], 1.0)[:, None]
    rel_e = dot(h_rel, rw_ref[...],
                (((1,), (1,)), ((), ()))) + rbb_ref[...][None, :]
    rel_range = (MARGIN + EPSILON) / OUT_DIM
    ph = rel_e[:, :IN_DIM] * jnp.float32(jnp.pi / rel_range)
    rtab_ref[...] = jnp.concatenate([jnp.cos(ph), jnp.sin(ph)], axis=1)


_k6 = pl.pallas_call(
    _k6_body,
    grid=(10,),
    in_specs=[
        pl.BlockSpec((1024, OUT_DIM), lambda i: (i, 0)),
        pl.BlockSpec((1024,), lambda i: (i,)),
        pl.BlockSpec((1024, OUT_DIM), lambda i: (i, 0)),
        pl.BlockSpec((1024,), lambda i: (i,)),
        pl.BlockSpec((1024,), lambda i: (i,)),
        pl.BlockSpec((OUT_DIM,), lambda i: (0,)),
        pl.BlockSpec((OUT_DIM,), lambda i: (0,)),
        pl.BlockSpec((OUT_DIM, OUT_DIM), lambda i: (0, 0)),
        pl.BlockSpec((OUT_DIM,), lambda i: (0,)),
        pl.BlockSpec((OUT_DIM, OUT_DIM), lambda i: (0, 0)),
        pl.BlockSpec((OUT_DIM,), lambda i: (0,)),
    ],
    out_specs=[
        pl.BlockSpec((1024, OUT_DIM), lambda i: (i, 0)),
        pl.BlockSpec((1024, OUT_DIM), lambda i: (i, 0)),
    ],
    out_shape=[_f32((TBL_ROWS, OUT_DIM)), _f32((TBL_ROWS, OUT_DIM))],
)


@functools.partial(
    pl.kernel,
    mesh=_mesh,
    out_type=_f32((NPAD,)),
    scratch_types=[
        pltpu.VMEM((CHUNK,), jnp.int32),
        pltpu.VMEM((CHUNK,), jnp.int32),
        pltpu.VMEM((CHUNK,), jnp.int32),
        pltpu.VMEM((_GB, OUT_DIM), jnp.float32),
        pltpu.VMEM((_GB, OUT_DIM), jnp.float32),
        pltpu.VMEM((_GB, OUT_DIM), jnp.float32),
        pltpu.VMEM((16,), jnp.float32),
        pltpu.SemaphoreType.DMA,
    ],
)
def _k7_rotate(s_hbm, d_hbm, r_hbm, ente_hbm, rtab_hbm, sc_out,
               s_v, d_v, r_v, hrows, trows, rrows, sbuf, sem):
    cid = lax.axis_index("c")
    sid = lax.axis_index("s")
    wid = cid * 16 + sid
    base = wid * CHUNK
    pltpu.sync_copy(s_hbm.at[pl.ds(base, CHUNK)], s_v)
    pltpu.sync_copy(d_hbm.at[pl.ds(base, CHUNK)], d_v)
    pltpu.sync_copy(r_hbm.at[pl.ds(base, CHUNK)], r_v)
    lanes = lax.iota(jnp.int32, 16)
    epsv = jnp.full((16,), 1e-12, jnp.float32)
    nb = CHUNK // _GB
    D2 = IN_DIM

    def batch(b, _):
        o = b * _GB
        h1 = pltpu.async_copy(ente_hbm.at[s_v.at[pl.ds(o, _GB)]], hrows, sem)
        h2 = pltpu.async_copy(ente_hbm.at[d_v.at[pl.ds(o, _GB)]], trows, sem)
        h3 = pltpu.async_copy(rtab_hbm.at[r_v.at[pl.ds(o, _GB)]], rrows, sem)
        h1.wait()
        h2.wait()
        h3.wait()

        def row(i, sv):
            acc = None
            for j in range(8):
                j16 = j * 16
                reh = hrows[i, pl.ds(j16, 16)]
                imh = hrows[i, pl.ds(D2 + j16, 16)]
                ret = trows[i, pl.ds(j16, 16)]
                imt = trows[i, pl.ds(D2 + j16, 16)]
                rc = rrows[i, pl.ds(j16, 16)]
                rs = rrows[i, pl.ds(D2 + j16, 16)]
                re_s = reh * rc - imh * rs - ret
                im_s = reh * rs + imh * rc - imt
                x = re_s * re_s + im_s * im_s + epsv
                ii = lax.bitcast_convert_type(x, jnp.int32)
                ii = 0x5F3759DF - lax.shift_right_logical(ii, 1)
                y = lax.bitcast_convert_type(ii, jnp.float32)
                y = y * (1.5 - 0.5 * x * y * y)
                y = y * (1.5 - 0.5 * x * y * y)
                t = x * y
                acc = t if acc is None else acc + t
            tot = _allsum(acc)
            return jnp.where(lanes == i, tot, sv)

        sv = lax.fori_loop(0, _GB, row, jnp.zeros((16,), jnp.float32))
        sbuf[pl.ds(0, 16)] = sv
        pltpu.sync_copy(sbuf, sc_out.at[pl.ds(base + o, 16)])
        return 0

    lax.fori_loop(0, nb, batch, 0)


def _k8_body(pos_ref, neg_ref, out_ref):
    t = jnp.maximum(pos_ref[...] - neg_ref[...] + jnp.float32(MARGIN), 0.0)
    s = jnp.sum(t) * jnp.float32(1.0 / (NEG_RATE * (N_TRIPLES // (NEG_RATE + 1))))
    out_ref[...] = jnp.full((1, 1), s, jnp.float32)


_k8 = pl.pallas_call(_k8_body, out_shape=_f32((1, 1)))


def kernel(triplets, ent_embed, rel_embed, a_W, a_b, a2_W, a2_b,
           bn0_gamma, bn0_beta, bn1_gamma, bn1_beta,
           entT_W, entT_b, relT_W, relT_b):
    trip = triplets.astype(jnp.int32)
    pad = jnp.full((NPAD - N_TRIPLES,), TRASH, jnp.int32)
    s_p = jnp.concatenate([trip[:, 0], pad])
    d_p = jnp.concatenate([trip[:, 1], pad])
    r_p = jnp.concatenate([trip[:, 2], pad])
    seg_p = jnp.concatenate([s_p, d_p])

    ones_stage = jnp.zeros((128, 128), jnp.float32).at[:, 0].set(1.0)
    zeros_hist = jnp.zeros((128, 128), jnp.float32)
    he, hr = _k1_hist(seg_p, r_p, ones_stage, zeros_hist)

    wp, biasp, mean1, cnt_r = _k2a(he, hr, ent_embed, rel_embed,
                                   a_W, a_b, bn0_gamma, bn0_beta)

    zpad = jnp.zeros((TBL_ROWS - N_ENT, IN_DIM), jnp.float32)
    e_pad = jnp.concatenate([ent_embed, zpad])
    r_pad = jnp.concatenate([rel_embed, zpad])
    p_ent, p3 = _k2b(e_pad, r_pad, wp)

    c_flat, ssqp = _k3_edges(s_p, d_p, r_p, p_ent, p3, biasp)

    npads = NPAD - N_TRIPLES
    ssq = ssqp.sum(axis=0) - 2.0 * npads * biasp * biasp
    var1 = ssq / jnp.float32(2 * N_TRIPLES) - mean1 * mean1
    alpha = bn1_gamma * lax.rsqrt(var1 + 1e-5)
    delta = bn1_beta - mean1 * alpha
    w2f = a2_W[0] * alpha
    u_const = jnp.dot(a2_W[0], delta) + a2_b[0]
    onehot = jnp.zeros((16,), jnp.float32).at[0].set(1.0)
    ucv = jnp.concatenate([jnp.full((16,), u_const, jnp.float32), onehot])

    zeros_acc = jnp.zeros((_SB, OUT_DIM // 2), jnp.float32)
    e_q = [k(c_flat, seg_p, w2f, ucv, zeros_acc) for k in _k5_ent]
    r_q = [k(c_flat, r_p, w2f, ucv, zeros_acc) for k in _k5_rel]

    def _stitch(qs, slab):
        return jnp.concatenate([qs[0][slab][0, :QUARTER],
                                qs[1][slab][0, :QUARTER],
                                qs[0][slab][1, :QUARTER],
                                qs[1][slab][1, :QUARTER]])

    def _assemble(qs):
        full = jnp.concatenate([_stitch(qs, 0), _stitch(qs, 1)], axis=1)
        return jnp.concatenate(
            [full, jnp.zeros((TBL_ROWS - 2 * HALF, OUT_DIM), jnp.float32)])

    hs = _assemble(e_q)
    rhs = _assemble(r_q)
    zpad1 = jnp.zeros((TBL_ROWS - 2 * HALF,), jnp.float32)
    ebs = jnp.concatenate([_stitch(e_q, 2)[:, 0], zpad1])
    ebr = jnp.concatenate([_stitch(r_q, 2)[:, 0], zpad1])

    ent_e, rtab = _k6(hs, ebs, rhs, ebr, cnt_r, alpha, delta,
                      entT_W, entT_b, relT_W, relT_b)

    scores = _k7_rotate(s_p, d_p, r_p, ent_e, rtab)

    k = N_TRIPLES // (NEG_RATE + 1)
    pos = jnp.broadcast_to(scores[:k][None, :], (NEG_RATE, k))
    neg = scores[k:N_TRIPLES].reshape(NEG_RATE, k)
    loss = _k8(pos, neg)
    return loss[0, 0]


# speedup vs baseline: 1.0184x; 1.0184x over previous
import functools

import numpy as np

import jax
import jax.numpy as jnp
from jax import lax
from jax.experimental import pallas as pl
from jax.experimental.pallas import tpu as pltpu
from jax.experimental.pallas import tpu_sc as plsc

N_ENT = 10000
N_REL = 10000
IN_DIM = 128
OUT_DIM = 256
MARGIN = 6.0
EPSILON = 2.0
NEG_RATE = 10
N_TRIPLES = 110000

NPAD = 110592
CHUNK = NPAD // 32
TRASH = 10000
TBL_ROWS = 10240
HALF = 5000
ACC_ROWS = 2560
QUARTER = 2500
ACC_W = 272

_mesh = plsc.VectorSubcoreMesh(core_axis_name="c", subcore_axis_name="s")


_GDN = lax.GatherDimensionNumbers(offset_dims=(), collapsed_slice_dims=(0,),
                                 start_index_map=(0,))


def _allsum(v):
    lanes = lax.iota(jnp.int32, 16)
    for k in (1, 2, 4, 8):
        p = (lanes ^ k).reshape(16, 1)
        v = v + lax.gather(v, p, _GDN, (1,),
                           mode=lax.GatherScatterMode.PROMISE_IN_BOUNDS)
    return v


def _f32(shape):
    return jax.ShapeDtypeStruct(shape, jnp.float32)


@functools.partial(
    pl.kernel,
    mesh=_mesh,
    out_type=[_f32((2, TBL_ROWS, 128)), _f32((2, TBL_ROWS, 128))],
    scratch_types=[
        pltpu.VMEM((128, 128), jnp.float32),
        pltpu.VMEM((128, 128), jnp.float32),
        pltpu.VMEM((128,), jnp.int32),
        pltpu.VMEM_SHARED((TBL_ROWS, 128), jnp.float32),
    ],
)
def _k1_hist(seg_hbm, rel_hbm, ones_hbm, zeros_hbm, out_e, out_r,
             stage_v, gbuf, idx_v, hist):
    cid = lax.axis_index("c")
    sid = lax.axis_index("s")
    wid = cid * 16 + sid
    lanes = lax.iota(jnp.int32, 16)
    pltpu.sync_copy(ones_hbm, stage_v)
    pltpu.sync_copy(zeros_hbm, gbuf)

    def ident(b, _):
        def w(k, _):
            idx_v[pl.ds(k * 16, 16)] = lanes + (sid * 640 + b * 128 + k * 16)
            return 0
        lax.fori_loop(0, 8, w, 0)
        return 0

    def run(src_hbm, base, nb, out):
        def zinit(b, _):
            ident(b, 0)
            pltpu.sync_copy(gbuf, hist.at[idx_v])
            return 0
        lax.fori_loop(0, 5, zinit, 0)
        plsc.subcore_barrier()

        def body(b, _):
            pltpu.sync_copy(src_hbm.at[pl.ds(base + b * 128, 128)], idx_v)
            pltpu.sync_copy(stage_v, hist.at[idx_v], add=True)
            return 0
        lax.fori_loop(0, nb, body, 0)
        plsc.subcore_barrier()

        def dump(b, _):
            ident(b, 0)
            pltpu.sync_copy(hist.at[idx_v], gbuf)
            pltpu.sync_copy(gbuf, out.at[cid, pl.ds(sid * 640 + b * 128, 128)])
            return 0
        lax.fori_loop(0, 5, dump, 0)
        plsc.subcore_barrier()
        pltpu.sync_copy(zeros_hbm, gbuf)

    run(seg_hbm, wid * (2 * CHUNK), (2 * CHUNK) // 128, out_e)
    run(rel_hbm, wid * CHUNK, CHUNK // 128, out_r)


def _k2a_body(he_ref, hr_ref, e_ref, r_ref, aw_ref, ab_ref, g0_ref, b0_ref,
              wp_ref, biasp_ref, mean1_ref, cntr_ref):
    he = he_ref[...]
    hr = hr_ref[...]
    cnt_e = he[0, :N_ENT, 0] + he[1, :N_ENT, 0]
    cnt_r_full = hr[0, :, 0] + hr[1, :, 0]
    cnt_r = cnt_r_full[:N_REL]
    E = e_ref[...]
    R = r_ref[...]
    dot = functools.partial(lax.dot_general,
                            preferred_element_type=jnp.float32,
                            precision=lax.Precision.HIGHEST)
    ce = cnt_e.reshape(1, N_ENT)
    cr = cnt_r.reshape(1, N_REL)
    sE = dot(ce, E, (((1,), (0,)), ((), ())))[0]
    qE = dot(ce, E * E, (((1,), (0,)), ((), ())))[0]
    qR = dot(cr, R * R, (((1,), (0,)), ((), ())))[0]
    n2 = jnp.float32(2 * N_TRIPLES)
    mE = sE / n2
    vE = qE / n2 - mE * mE
    vR = qR / jnp.float32(N_TRIPLES)
    m = jnp.concatenate([mE, mE, jnp.zeros_like(mE)])
    v = jnp.concatenate([vE, vE, vR])
    g0 = g0_ref[...]
    b0 = b0_ref[...]
    scale0 = g0 * lax.rsqrt(v + 1e-5)
    shift0 = b0 - m * scale0
    aw = aw_ref[...]
    wp_ref[...] = aw * scale0[None, :]
    biasp_ref[...] = dot(aw, shift0.reshape(3 * IN_DIM, 1),
                         (((1,), (0,)), ((), ())))[:, 0] + ab_ref[...]
    mean1_ref[...] = dot(aw, b0.reshape(3 * IN_DIM, 1),
                         (((1,), (0,)), ((), ())))[:, 0] + ab_ref[...]
    cntr_ref[...] = cnt_r_full


_k2a = pl.pallas_call(
    _k2a_body,
    out_shape=[_f32((OUT_DIM, 3 * IN_DIM)), _f32((OUT_DIM,)),
               _f32((OUT_DIM,)), _f32((TBL_ROWS,))],
)


def _k2b_body(e_ref, r_ref, wp_ref, pent_ref, p3_ref):
    wp = wp_ref[...]
    w1 = wp[:, :IN_DIM]
    w2 = wp[:, IN_DIM:2 * IN_DIM]
    w3 = wp[:, 2 * IN_DIM:]
    dot = functools.partial(lax.dot_general,
                            preferred_element_type=jnp.float32,
                            precision=lax.Precision.HIGHEST)
    E = e_ref[...]
    p1 = dot(E, w1, (((1,), (1,)), ((), ())))
    p2 = dot(E, w2, (((1,), (1,)), ((), ())))
    pent_ref[...] = jnp.concatenate([p1, p2], axis=1)
    p3_ref[...] = dot(r_ref[...], w3, (((1,), (1,)), ((), ())))


_k2b = pl.pallas_call(
    _k2b_body,
    grid=(10,),
    in_specs=[
        pl.BlockSpec((1024, IN_DIM), lambda i: (i, 0)),
        pl.BlockSpec((1024, IN_DIM), lambda i: (i, 0)),
        pl.BlockSpec((OUT_DIM, 3 * IN_DIM), lambda i: (0, 0)),
    ],
    out_specs=[
        pl.BlockSpec((1024, 2 * OUT_DIM), lambda i: (i, 0)),
        pl.BlockSpec((1024, OUT_DIM), lambda i: (i, 0)),
    ],
    out_shape=[_f32((TBL_ROWS, 2 * OUT_DIM)), _f32((TBL_ROWS, OUT_DIM))],
)


_GB3 = 32
_GB = 16


@functools.partial(
    pl.kernel,
    mesh=_mesh,
    out_type=[_f32((2 * NPAD * OUT_DIM,)), _f32((32, OUT_DIM))],
    scratch_types=[
        pltpu.VMEM((CHUNK,), jnp.int32),
        pltpu.VMEM((CHUNK,), jnp.int32),
        pltpu.VMEM((CHUNK,), jnp.int32),
        pltpu.VMEM((_GB3, 2 * OUT_DIM), jnp.float32),
        pltpu.VMEM((_GB3, 2 * OUT_DIM), jnp.float32),
        pltpu.VMEM((_GB3, OUT_DIM), jnp.float32),
        pltpu.VMEM((_GB3 * OUT_DIM,), jnp.float32),
        pltpu.VMEM((_GB3 * OUT_DIM,), jnp.float32),
        pltpu.VMEM((OUT_DIM,), jnp.float32),
        pltpu.VMEM((OUT_DIM,), jnp.float32),
        pltpu.SemaphoreType.DMA,
    ],
)
def _k3_edges(s_hbm, d_hbm, r_hbm, pent_hbm, p3_hbm, biasp_hbm,
              c_out, ssq_out, s_v, d_v, r_v, srows, drows, rrows,
              c1b, c2b, bias_v, ssq_v, sem):
    cid = lax.axis_index("c")
    sid = lax.axis_index("s")
    wid = cid * 16 + sid
    base = wid * CHUNK
    pltpu.sync_copy(s_hbm.at[pl.ds(base, CHUNK)], s_v)
    pltpu.sync_copy(d_hbm.at[pl.ds(base, CHUNK)], d_v)
    pltpu.sync_copy(r_hbm.at[pl.ds(base, CHUNK)], r_v)
    pltpu.sync_copy(biasp_hbm, bias_v)
    biasvs = [bias_v[pl.ds(j * 16, 16)] for j in range(16)]
    nb = CHUNK // _GB3

    def batch(b, acc):
        o = b * _GB3
        h1 = pltpu.async_copy(pent_hbm.at[s_v.at[pl.ds(o, _GB3)]], srows, sem)
        h2 = pltpu.async_copy(pent_hbm.at[d_v.at[pl.ds(o, _GB3)]], drows, sem)
        h3 = pltpu.async_copy(p3_hbm.at[r_v.at[pl.ds(o, _GB3)]], rrows, sem)
        h1.wait()
        h2.wait()
        h3.wait()

        def row(i, acc):
            acc = list(acc)
            for j in range(16):
                j16 = j * 16
                s_lo = srows[i, pl.ds(j16, 16)]
                s_hi = srows[i, pl.ds(OUT_DIM + j16, 16)]
                d_lo = drows[i, pl.ds(j16, 16)]
                d_hi = drows[i, pl.ds(OUT_DIM + j16, 16)]
                rr = rrows[i, pl.ds(j16, 16)]
                c1 = s_lo + d_hi + rr + biasvs[j]
                c2 = d_lo + s_hi - rr + biasvs[j]
                c1b[pl.ds(i * OUT_DIM + j16, 16)] = c1
                c2b[pl.ds(i * OUT_DIM + j16, 16)] = c2
                acc[j] = acc[j] + c1 * c1 + c2 * c2
            return tuple(acc)

        acc = lax.fori_loop(0, _GB3, row, acc)
        off = (base + o) * OUT_DIM
        pltpu.sync_copy(c1b, c_out.at[pl.ds(off, _GB3 * OUT_DIM)])
        pltpu.sync_copy(c2b, c_out.at[pl.ds(NPAD * OUT_DIM + off,
                                            _GB3 * OUT_DIM)])
        return acc

    zero = jnp.zeros((16,), jnp.float32)
    acc = lax.fori_loop(0, nb, batch, (zero,) * 16)
    for j in range(16):
        ssq_v[pl.ds(j * 16, 16)] = acc[j]
    pltpu.sync_copy(ssq_v, ssq_out.at[wid])


_SB = 32


def _make_k5(total_rows, qidx):
    nb = total_rows // 16 // _SB
    hw = OUT_DIM // 2
    dump = ACC_ROWS // 16

    @functools.partial(
        pl.kernel,
        mesh=_mesh,
        out_type=[_f32((2, ACC_ROWS, hw)), _f32((2, ACC_ROWS, hw)),
                  _f32((2, ACC_ROWS, hw))],
        scratch_types=[
            pltpu.VMEM((_SB * OUT_DIM,), jnp.float32),
            pltpu.VMEM((_SB, hw), jnp.float32),
            pltpu.VMEM((_SB, hw), jnp.float32),
            pltpu.VMEM((_SB, hw), jnp.float32),
            pltpu.VMEM((_SB, hw), jnp.float32),
            pltpu.VMEM((_SB,), jnp.int32),
            pltpu.VMEM((_SB,), jnp.int32),
            pltpu.VMEM((OUT_DIM,), jnp.float32),
            pltpu.VMEM((32,), jnp.float32),
            pltpu.VMEM_SHARED((ACC_ROWS, hw), jnp.float32),
            pltpu.VMEM_SHARED((ACC_ROWS, hw), jnp.float32),
            pltpu.VMEM_SHARED((ACC_ROWS, hw), jnp.float32),
        ],
    )
    def k5(c_hbm, seg_hbm, w2f_hbm, ucv_hbm, zeros_hbm, out_a, out_b, out_c,
           cbuf, stga, stgb, stgc, zbuf, segraw, idx_v, w_v, uc_v,
           acca, accb, accc):
        cid = lax.axis_index("c")
        sid = lax.axis_index("s")
        lanes = lax.iota(jnp.int32, 16)
        pltpu.sync_copy(w2f_hbm, w_v)
        pltpu.sync_copy(ucv_hbm, uc_v)
        pltpu.sync_copy(zeros_hbm, zbuf)
        pltpu.sync_copy(zeros_hbm, stgc)
        lane0 = uc_v[pl.ds(16, 16)]

        def ident(b, _):
            def w(k, _):
                idx_v[pl.ds(k * 16, 16)] = lanes + (sid * dump + b * _SB
                                                    + k * 16)
                return 0
            lax.fori_loop(0, _SB // 16, w, 0)
            return 0

        def zinit(b, _):
            ident(b, 0)
            pltpu.sync_copy(zbuf, acca.at[idx_v])
            pltpu.sync_copy(zbuf, accb.at[idx_v])
            pltpu.sync_copy(zbuf, accc.at[idx_v])
            return 0
        lax.fori_loop(0, dump // _SB, zinit, 0)
        plsc.subcore_barrier()
        ws = [w_v[pl.ds(j * 16, 16)] for j in range(16)]
        ucv = uc_v[pl.ds(0, 16)]
        lo = cid * HALF + qidx * QUARTER
        rows_per = total_rows // 16

        def batch(b, _):
            row0 = sid * rows_per + b * _SB
            pltpu.sync_copy(c_hbm.at[pl.ds(row0 * OUT_DIM,
                                           _SB * OUT_DIM)], cbuf)
            pltpu.sync_copy(seg_hbm.at[pl.ds(row0, _SB)], segraw)
            for k in range(_SB // 16):
                vseg = segraw[pl.ds(k * 16, 16)] - lo
                ok = (vseg >= 0) & (vseg < QUARTER)
                idx_v[pl.ds(k * 16, 16)] = jnp.where(ok, vseg, QUARTER)

            for i in range(_SB):
                cs = [cbuf[pl.ds(i * OUT_DIM + j * 16, 16)]
                      for j in range(16)]
                ua = cs[0] * ws[0]
                for j in range(1, 16):
                    ua = ua + cs[j] * ws[j]
                u = _allsum(ua) + ucv
                lk = jnp.maximum(u, 0.0) + 0.01 * jnp.minimum(u, 0.0)
                e = jnp.exp(-lk)
                for j in range(8):
                    stga[i, pl.ds(j * 16, 16)] = cs[j] * e
                for j in range(8):
                    stgb[i, pl.ds(j * 16, 16)] = cs[8 + j] * e
                stgc[i, pl.ds(0, 16)] = e * lane0

            pltpu.sync_copy(stga, acca.at[idx_v], add=True)
            pltpu.sync_copy(stgb, accb.at[idx_v], add=True)
            pltpu.sync_copy(stgc, accc.at[idx_v], add=True)
            return 0

        lax.fori_loop(0, nb, batch, 0)
        plsc.subcore_barrier()

        def dumploop(b, _):
            ident(b, 0)
            osl = pl.ds(sid * dump + b * _SB, _SB)
            pltpu.sync_copy(acca.at[idx_v], zbuf)
            pltpu.sync_copy(zbuf, out_a.at[cid, osl])
            pltpu.sync_copy(accb.at[idx_v], zbuf)
            pltpu.sync_copy(zbuf, out_b.at[cid, osl])
            pltpu.sync_copy(accc.at[idx_v], zbuf)
            pltpu.sync_copy(zbuf, out_c.at[cid, osl])
            return 0
        lax.fori_loop(0, dump // _SB, dumploop, 0)

    return k5


_k5_ent = [_make_k5(2 * NPAD, q) for q in (0, 1)]
_k5_rel = [_make_k5(NPAD, q) for q in (0, 1)]


def _k6_body(hs_ref, ebs_ref, rhs_ref, ebr_ref, cntr_ref, al_ref, de_ref,
             ew_ref, ebb_ref, rw_ref, rbb_ref, ente_ref, rtab_ref):
    alpha = al_ref[...]
    delta = de_ref[...]
    dot = functools.partial(lax.dot_general,
                            preferred_element_type=jnp.float32,
                            precision=lax.Precision.HIGHEST)
    ebs = ebs_ref[...]
    num = alpha[None, :] * hs_ref[...] + delta[None, :] * ebs[:, None]
    h_ent = num / jnp.where(ebs == 0.0, 1e-12, ebs)[:, None]
    ente_ref[...] = dot(h_ent, ew_ref[...],
                        (((1,), (1,)), ((), ()))) + ebb_ref[...][None, :]
    ebr = ebr_ref[...]
    rnum = alpha[None, :] * rhs_ref[...] + delta[None, :] * ebr[:, None]
    h_rel = rnum / jnp.maximum(cntr_ref[...], 1.0)[:, None]
    rel_e = dot(h_rel, rw_ref[...],
                (((1,), (1,)), ((), ()))) + rbb_ref[...][None, :]
    rel_range = (MARGIN + EPSILON) / OUT_DIM
    ph = rel_e[:, :IN_DIM] * jnp.float32(jnp.pi / rel_range)
    rtab_ref[...] = jnp.concatenate([jnp.cos(ph), jnp.sin(ph)], axis=1)


_k6 = pl.pallas_call(
    _k6_body,
    grid=(10,),
    in_specs=[
        pl.BlockSpec((1024, OUT_DIM), lambda i: (i, 0)),
        pl.BlockSpec((1024,), lambda i: (i,)),
        pl.BlockSpec((1024, OUT_DIM), lambda i: (i, 0)),
        pl.BlockSpec((1024,), lambda i: (i,)),
        pl.BlockSpec((1024,), lambda i: (i,)),
        pl.BlockSpec((OUT_DIM,), lambda i: (0,)),
        pl.BlockSpec((OUT_DIM,), lambda i: (0,)),
        pl.BlockSpec((OUT_DIM, OUT_DIM), lambda i: (0, 0)),
        pl.BlockSpec((OUT_DIM,), lambda i: (0,)),
        pl.BlockSpec((OUT_DIM, OUT_DIM), lambda i: (0, 0)),
        pl.BlockSpec((OUT_DIM,), lambda i: (0,)),
    ],
    out_specs=[
        pl.BlockSpec((1024, OUT_DIM), lambda i: (i, 0)),
        pl.BlockSpec((1024, OUT_DIM), lambda i: (i, 0)),
    ],
    out_shape=[_f32((TBL_ROWS, OUT_DIM)), _f32((TBL_ROWS, OUT_DIM))],
)


@functools.partial(
    pl.kernel,
    mesh=_mesh,
    out_type=_f32((NPAD,)),
    scratch_types=[
        pltpu.VMEM((CHUNK,), jnp.int32),
        pltpu.VMEM((CHUNK,), jnp.int32),
        pltpu.VMEM((CHUNK,), jnp.int32),
        pltpu.VMEM((_GB, OUT_DIM), jnp.float32),
        pltpu.VMEM((_GB, OUT_DIM), jnp.float32),
        pltpu.VMEM((_GB, OUT_DIM), jnp.float32),
        pltpu.VMEM((16,), jnp.float32),
        pltpu.SemaphoreType.DMA,
    ],
)
def _k7_rotate(s_hbm, d_hbm, r_hbm, ente_hbm, rtab_hbm, sc_out,
               s_v, d_v, r_v, hrows, trows, rrows, sbuf, sem):
    cid = lax.axis_index("c")
    sid = lax.axis_index("s")
    wid = cid * 16 + sid
    base = wid * CHUNK
    pltpu.sync_copy(s_hbm.at[pl.ds(base, CHUNK)], s_v)
    pltpu.sync_copy(d_hbm.at[pl.ds(base, CHUNK)], d_v)
    pltpu.sync_copy(r_hbm.at[pl.ds(base, CHUNK)], r_v)
    lanes = lax.iota(jnp.int32, 16)
    epsv = jnp.full((16,), 1e-12, jnp.float32)
    nb = CHUNK // _GB
    D2 = IN_DIM

    def batch(b, _):
        o = b * _GB
        h1 = pltpu.async_copy(ente_hbm.at[s_v.at[pl.ds(o, _GB)]], hrows, sem)
        h2 = pltpu.async_copy(ente_hbm.at[d_v.at[pl.ds(o, _GB)]], trows, sem)
        h3 = pltpu.async_copy(rtab_hbm.at[r_v.at[pl.ds(o, _GB)]], rrows, sem)
        h1.wait()
        h2.wait()
        h3.wait()

        def row(i, sv):
            acc = None
            for j in range(8):
                j16 = j * 16
                reh = hrows[i, pl.ds(j16, 16)]
                imh = hrows[i, pl.ds(D2 + j16, 16)]
                ret = trows[i, pl.ds(j16, 16)]
                imt = trows[i, pl.ds(D2 + j16, 16)]
                rc = rrows[i, pl.ds(j16, 16)]
                rs = rrows[i, pl.ds(D2 + j16, 16)]
                re_s = reh * rc - imh * rs - ret
                im_s = reh * rs + imh * rc - imt
                x = re_s * re_s + im_s * im_s + epsv
                ii = lax.bitcast_convert_type(x, jnp.int32)
                ii = 0x5F3759DF - lax.shift_right_logical(ii, 1)
                y = lax.bitcast_convert_type(ii, jnp.float32)
                y = y * (1.5 - 0.5 * x * y * y)
                y = y * (1.5 - 0.5 * x * y * y)
                t = x * y
                acc = t if acc is None else acc + t
            tot = _allsum(acc)
            return jnp.where(lanes == i, tot, sv)

        sv = lax.fori_loop(0, _GB, row, jnp.zeros((16,), jnp.float32))
        sbuf[pl.ds(0, 16)] = sv
        pltpu.sync_copy(sbuf, sc_out.at[pl.ds(base + o, 16)])
        return 0

    lax.fori_loop(0, nb, batch, 0)


def _k8_body(pos_ref, neg_ref, out_ref):
    t = jnp.maximum(pos_ref[...] - neg_ref[...] + jnp.float32(MARGIN), 0.0)
    s = jnp.sum(t) * jnp.float32(1.0 / (NEG_RATE * (N_TRIPLES // (NEG_RATE + 1))))
    out_ref[...] = jnp.full((1, 1), s, jnp.float32)


_k8 = pl.pallas_call(_k8_body, out_shape=_f32((1, 1)))


def kernel(triplets, ent_embed, rel_embed, a_W, a_b, a2_W, a2_b,
           bn0_gamma, bn0_beta, bn1_gamma, bn1_beta,
           entT_W, entT_b, relT_W, relT_b):
    trip = triplets.astype(jnp.int32)
    pad = jnp.full((NPAD - N_TRIPLES,), TRASH, jnp.int32)
    s_p = jnp.concatenate([trip[:, 0], pad])
    d_p = jnp.concatenate([trip[:, 1], pad])
    r_p = jnp.concatenate([trip[:, 2], pad])
    seg_p = jnp.concatenate([s_p, d_p])

    ones_stage = jnp.zeros((128, 128), jnp.float32).at[:, 0].set(1.0)
    zeros_hist = jnp.zeros((128, 128), jnp.float32)
    he, hr = _k1_hist(seg_p, r_p, ones_stage, zeros_hist)

    wp, biasp, mean1, cnt_r = _k2a(he, hr, ent_embed, rel_embed,
                                   a_W, a_b, bn0_gamma, bn0_beta)

    zpad = jnp.zeros((TBL_ROWS - N_ENT, IN_DIM), jnp.float32)
    e_pad = jnp.concatenate([ent_embed, zpad])
    r_pad = jnp.concatenate([rel_embed, zpad])
    p_ent, p3 = _k2b(e_pad, r_pad, wp)

    c_flat, ssqp = _k3_edges(s_p, d_p, r_p, p_ent, p3, biasp)

    npads = NPAD - N_TRIPLES
    ssq = ssqp.sum(axis=0) - 2.0 * npads * biasp * biasp
    var1 = ssq / jnp.float32(2 * N_TRIPLES) - mean1 * mean1
    alpha = bn1_gamma * lax.rsqrt(var1 + 1e-5)
    delta = bn1_beta - mean1 * alpha
    w2f = a2_W[0] * alpha
    u_const = jnp.dot(a2_W[0], delta) + a2_b[0]
    onehot = jnp.zeros((16,), jnp.float32).at[0].set(1.0)
    ucv = jnp.concatenate([jnp.full((16,), u_const, jnp.float32), onehot])

    zeros_acc = jnp.zeros((_SB, OUT_DIM // 2), jnp.float32)
    e_q = [k(c_flat, seg_p, w2f, ucv, zeros_acc) for k in _k5_ent]
    r_q = [k(c_flat, r_p, w2f, ucv, zeros_acc) for k in _k5_rel]

    def _stitch(qs, slab):
        return jnp.concatenate([qs[0][slab][0, :QUARTER],
                                qs[1][slab][0, :QUARTER],
                                qs[0][slab][1, :QUARTER],
                                qs[1][slab][1, :QUARTER]])

    def _assemble(qs):
        full = jnp.concatenate([_stitch(qs, 0), _stitch(qs, 1)], axis=1)
        return jnp.concatenate(
            [full, jnp.zeros((TBL_ROWS - 2 * HALF, OUT_DIM), jnp.float32)])

    hs = _assemble(e_q)
    rhs = _assemble(r_q)
    zpad1 = jnp.zeros((TBL_ROWS - 2 * HALF,), jnp.float32)
    ebs = jnp.concatenate([_stitch(e_q, 2)[:, 0], zpad1])
    ebr = jnp.concatenate([_stitch(r_q, 2)[:, 0], zpad1])

    ent_e, rtab = _k6(hs, ebs, rhs, ebr, cnt_r, alpha, delta,
                      entT_W, entT_b, relT_W, relT_b)

    scores = _k7_rotate(s_p, d_p, r_p, ent_e, rtab)

    k = N_TRIPLES // (NEG_RATE + 1)
    pos = jnp.broadcast_to(scores[:k][None, :], (NEG_RATE, k))
    neg = scores[k:N_TRIPLES].reshape(NEG_RATE, k)
    loss = _k8(pos, neg)
    return loss[0, 0]
